# baseline (speedup 1.0000x reference)
import dataclasses

import jax
import jax.numpy as jnp
from jax import lax
from jax.experimental import pallas as pl
from jax.experimental.pallas import tpu as pltpu
from jax.experimental.pallas import tpu_sc as plsc

N_NODES = 10000
FEAT = 128
N_EDGES = 320000

NC = 2
NS = 16
LANES = 16

CHUNK = 128
K_CHUNKS = 80
E_PAD = CHUNK * K_CHUNKS * NC * NS
ACC_N = 10240
ROWS_PER_SUBCORE = ACC_N // NS
ZB_ROWS = 64


def _matmul_body(f_ref, wt_ref, o_ref):
    o_ref[...] = jnp.dot(f_ref[...], wt_ref[...],
                         preferred_element_type=jnp.float32)


def _matmul(feat, Wt):
    blk = 1000
    return pl.pallas_call(
        _matmul_body,
        grid=(N_NODES // blk,),
        in_specs=[
            pl.BlockSpec((blk, FEAT), lambda i: (i, 0)),
            pl.BlockSpec((FEAT, FEAT), lambda i: (0, 0)),
        ],
        out_specs=pl.BlockSpec((blk, FEAT), lambda i: (i, 0)),
        out_shape=jax.ShapeDtypeStruct((N_NODES, FEAT), jnp.float32),
    )(feat, Wt)


def _edge_body(h_hbm, epk_hbm, out_hbm,
               ib0, ib1, mb, scol, zbv, acc,
               si0, si1, sg, ss):
    core = lax.axis_index("c")
    sid = lax.axis_index("s")
    wid = core * NS + sid
    cbase = wid * K_CHUNKS

    ib = (ib0, ib1)
    si = (si0, si1)

    pltpu.async_copy(epk_hbm.at[cbase], ib0, si0)

    @pl.loop(0, ZB_ROWS)
    def _(i):
        @pl.loop(0, FEAT, step=LANES)
        def _(j):
            zbv[i, pl.ds(j, LANES)] = jnp.zeros((LANES,), jnp.float32)

    @pl.loop(0, ROWS_PER_SUBCORE, step=ZB_ROWS)
    def _(r):
        pltpu.sync_copy(zbv, acc.at[pl.ds(sid * ROWS_PER_SUBCORE + r, ZB_ROWS)])

    plsc.subcore_barrier()

    def wait_idx(g, m):
        pltpu.make_async_copy(epk_hbm.at[cbase + g], ib[m], si[m]).wait()

    def wait_msg_bytes(sem):
        pltpu.make_async_copy(h_hbm.at[pl.ds(0, CHUNK)], mb, sem).wait()

    def step(g, m):
        wait_idx(g, m)
        pltpu.async_copy(h_hbm.at[ib[m].at[0]], mb, sg)

        @pl.when(g + 1 < K_CHUNKS)
        def _():
            pltpu.async_copy(epk_hbm.at[cbase + g + 1], ib[1 - m], si[1 - m])

        for k in range(CHUNK // LANES):
            sl = pl.ds(k * LANES, LANES)
            scol[sl] = ib[m][1, sl]

        wait_msg_bytes(sg)

        @pl.loop(0, CHUNK, step=LANES)
        def _(e0):
            w16 = plsc.bitcast(ib[m][2, pl.ds(e0, LANES)], jnp.float32)
            for l in range(LANES):
                wvec = jnp.full((LANES,), w16[l], jnp.float32)
                for j in range(FEAT // LANES):
                    sl = pl.ds(j * LANES, LANES)
                    mb[e0 + l, sl] = mb[e0 + l, sl] * wvec

        pltpu.async_copy(mb, acc.at[scol], ss, add=True)
        wait_msg_bytes(ss)

    @pl.loop(0, K_CHUNKS, step=2)
    def _(g):
        step(g, 0)
        step(g + 1, 1)

    plsc.subcore_barrier()

    @pl.loop(0, ROWS_PER_SUBCORE, step=ZB_ROWS)
    def _(r):
        rr = sid * ROWS_PER_SUBCORE + r
        pltpu.sync_copy(acc.at[pl.ds(rr, ZB_ROWS)],
                        out_hbm.at[core, pl.ds(rr, ZB_ROWS)])


def _edge_scatter(h, epk):
    mesh = plsc.VectorSubcoreMesh(core_axis_name="c", subcore_axis_name="s")
    cp = pltpu.CompilerParams()
    if "needs_layout_passes" in pltpu.CompilerParams.__dataclass_fields__:
        cp = dataclasses.replace(cp, needs_layout_passes=False)
    kern = pl.kernel(
        _edge_body,
        compiler_params=cp,
        out_type=jax.ShapeDtypeStruct((NC, ACC_N, FEAT), jnp.float32),
        mesh=mesh,
        scratch_types=[
            pltpu.VMEM((3, CHUNK), jnp.int32),
            pltpu.VMEM((3, CHUNK), jnp.int32),
            pltpu.VMEM((CHUNK, FEAT), jnp.float32),
            pltpu.VMEM((CHUNK,), jnp.int32),
            pltpu.VMEM((ZB_ROWS, FEAT), jnp.float32),
            pltpu.VMEM_SHARED((ACC_N, FEAT), jnp.float32),
            pltpu.SemaphoreType.DMA,
            pltpu.SemaphoreType.DMA,
            pltpu.SemaphoreType.DMA,
            pltpu.SemaphoreType.DMA,
        ],
    )
    return kern(h, epk)


def _combine_body(p_ref, a_ref, o_ref):
    s = p_ref[0] + p_ref[1]
    o_ref[...] = jnp.where(s >= 0, s, a_ref[0] * s)


def _combine(partial, prelu_w):
    blk = 1000
    return pl.pallas_call(
        _combine_body,
        grid=(N_NODES // blk,),
        in_specs=[
            pl.BlockSpec((NC, blk, FEAT), lambda i: (0, i, 0)),
            pl.BlockSpec(memory_space=pltpu.SMEM),
        ],
        out_specs=pl.BlockSpec((blk, FEAT), lambda i: (i, 0)),
        out_shape=jax.ShapeDtypeStruct((N_NODES, FEAT), jnp.float32),
    )(partial, prelu_w.reshape(1))


def kernel(feat, edge_index, edge_weight, W, prelu_w):
    row = edge_index[0].astype(jnp.int32)
    col = edge_index[1].astype(jnp.int32)
    pad = E_PAD - N_EDGES
    row = jnp.pad(row, (0, pad))
    col = jnp.pad(col, (0, pad))
    ew = jnp.pad(edge_weight.astype(jnp.float32), (0, pad))
    epk = jnp.stack([
        row.reshape(-1, CHUNK),
        col.reshape(-1, CHUNK),
        lax.bitcast_convert_type(ew, jnp.int32).reshape(-1, CHUNK),
    ], axis=1)

    h = _matmul(feat, W.T)
    partial = _edge_scatter(h, epk)
    return _combine(partial, prelu_w)


# speedup vs baseline: 1.2229x; 1.2229x over previous
import dataclasses

import jax
import jax.numpy as jnp
from jax import lax
from jax.experimental import pallas as pl
from jax.experimental.pallas import tpu as pltpu
from jax.experimental.pallas import tpu_sc as plsc

N_NODES = 10000
FEAT = 128
N_EDGES = 320000

NC = 2
NS = 16
LANES = 16

CHUNK = 64
K_CHUNKS = 160
NBUF = 5
LAG_G = 3
LAG_I = 4
E_PAD = CHUNK * K_CHUNKS * NC * NS
ACC_N = 10112
ROWS_PER_SUBCORE = ACC_N // NS


def _matmul_body(f_ref, wt_ref, o_ref):
    o_ref[...] = jnp.dot(f_ref[...], wt_ref[...],
                         preferred_element_type=jnp.float32)


def _matmul(feat, Wt):
    blk = 1000
    return pl.pallas_call(
        _matmul_body,
        grid=(N_NODES // blk,),
        in_specs=[
            pl.BlockSpec((blk, FEAT), lambda i: (i, 0)),
            pl.BlockSpec((FEAT, FEAT), lambda i: (0, 0)),
        ],
        out_specs=pl.BlockSpec((blk, FEAT), lambda i: (i, 0)),
        out_shape=jax.ShapeDtypeStruct((N_NODES, FEAT), jnp.float32),
    )(feat, Wt)


def _edge_body(h_hbm, epk_hbm, out_hbm,
               ib0, ib1, ib2, ib3, ib4,
               mb0, mb1, mb2, mb3, mb4,
               sc0, sc1, sc2, sc3, sc4,
               acc,
               si0, si1, si2, si3, si4,
               sg0, sg1, sg2, sg3, sg4,
               ss0, ss1, ss2, ss3, ss4):
    core = lax.axis_index("c")
    sid = lax.axis_index("s")
    wid = core * NS + sid
    cbase = wid * K_CHUNKS

    ib = (ib0, ib1, ib2, ib3, ib4)
    mb = (mb0, mb1, mb2, mb3, mb4)
    scol = (sc0, sc1, sc2, sc3, sc4)
    si = (si0, si1, si2, si3, si4)
    sg = (sg0, sg1, sg2, sg3, sg4)
    ss = (ss0, ss1, ss2, ss3, ss4)

    def fetch_idx(g, m):
        pltpu.async_copy(epk_hbm.at[cbase + g], ib[m], si[m])

    def wait_idx(g, m):
        pltpu.make_async_copy(epk_hbm.at[cbase + g], ib[m], si[m]).wait()

    def issue_gather(m):
        pltpu.async_copy(h_hbm.at[ib[m].at[0]], mb[m], sg[m])

    def wait_msg_bytes(m, sem):
        pltpu.make_async_copy(h_hbm.at[pl.ds(0, CHUNK)], mb[m], sem[m]).wait()

    @pl.loop(0, CHUNK)
    def _(i):
        @pl.loop(0, FEAT, step=LANES)
        def _(j):
            mb0[i, pl.ds(j, LANES)] = jnp.zeros((LANES,), jnp.float32)

    rbase = sid * ROWS_PER_SUBCORE

    @pl.loop(0, 576, step=CHUNK)
    def _(r):
        pltpu.sync_copy(mb0, acc.at[pl.ds(rbase + r, CHUNK)])

    pltpu.sync_copy(mb0.at[pl.ds(0, 56)], acc.at[pl.ds(rbase + 576, 56)])

    for k in range(LAG_I):
        fetch_idx(k, k)
    for k in range(LAG_G):
        wait_idx(k, k)
        issue_gather(k)

    plsc.subcore_barrier()

    def step(g, m):
        m2 = (m + LAG_I) % NBUF

        @pl.when(g + LAG_I < K_CHUNKS)
        def _():
            @pl.when(g + LAG_I >= NBUF)
            def _():
                wait_msg_bytes(m2, ss)
            fetch_idx(g + LAG_I, m2)

        m3 = (m + LAG_G) % NBUF

        @pl.when(g + LAG_G < K_CHUNKS)
        def _():
            wait_idx(g + LAG_G, m3)
            issue_gather(m3)

        wait_msg_bytes(m, sg)

        for k in range(CHUNK // LANES):
            sl = pl.ds(k * LANES, LANES)
            scol[m][sl] = ib[m][1, sl]

        @pl.loop(0, CHUNK, step=LANES)
        def _(e0):
            w16 = plsc.bitcast(ib[m][2, pl.ds(e0, LANES)], jnp.float32)
            for l in range(LANES):
                wvec = jnp.full((LANES,), w16[l], jnp.float32)
                for j in range(FEAT // LANES):
                    sl = pl.ds(j * LANES, LANES)
                    mb[m][e0 + l, sl] = mb[m][e0 + l, sl] * wvec

        pltpu.async_copy(mb[m], acc.at[scol[m]], ss[m], add=True)

    @pl.loop(0, K_CHUNKS, step=NBUF)
    def _(g):
        for k in range(NBUF):
            step(g + k, k)

    for m in range(NBUF):
        wait_msg_bytes(m, ss)
    plsc.subcore_barrier()

    @pl.loop(0, 576, step=CHUNK)
    def _(r):
        pltpu.sync_copy(acc.at[pl.ds(rbase + r, CHUNK)],
                        out_hbm.at[core, pl.ds(rbase + r, CHUNK)])

    pltpu.sync_copy(acc.at[pl.ds(rbase + 576, 56)],
                    out_hbm.at[core, pl.ds(rbase + 576, 56)])


def _edge_scatter(h, epk):
    mesh = plsc.VectorSubcoreMesh(core_axis_name="c", subcore_axis_name="s")
    cp = pltpu.CompilerParams()
    if "needs_layout_passes" in pltpu.CompilerParams.__dataclass_fields__:
        cp = dataclasses.replace(cp, needs_layout_passes=False)
    scratch = (
        [pltpu.VMEM((3, CHUNK), jnp.int32) for _ in range(NBUF)]
        + [pltpu.VMEM((CHUNK, FEAT), jnp.float32) for _ in range(NBUF)]
        + [pltpu.VMEM((CHUNK,), jnp.int32) for _ in range(NBUF)]
        + [pltpu.VMEM_SHARED((ACC_N, FEAT), jnp.float32)]
        + [pltpu.SemaphoreType.DMA] * (3 * NBUF)
    )
    kern = pl.kernel(
        _edge_body,
        compiler_params=cp,
        out_type=jax.ShapeDtypeStruct((NC, ACC_N, FEAT), jnp.float32),
        mesh=mesh,
        scratch_types=scratch,
    )
    return kern(h, epk)


def _combine_body(p_ref, a_ref, o_ref):
    s = p_ref[0] + p_ref[1]
    o_ref[...] = jnp.where(s >= 0, s, a_ref[0] * s)


def _combine(partial, prelu_w):
    blk = 1000
    return pl.pallas_call(
        _combine_body,
        grid=(N_NODES // blk,),
        in_specs=[
            pl.BlockSpec((NC, blk, FEAT), lambda i: (0, i, 0)),
            pl.BlockSpec(memory_space=pltpu.SMEM),
        ],
        out_specs=pl.BlockSpec((blk, FEAT), lambda i: (i, 0)),
        out_shape=jax.ShapeDtypeStruct((N_NODES, FEAT), jnp.float32),
    )(partial, prelu_w.reshape(1))


def kernel(feat, edge_index, edge_weight, W, prelu_w):
    row = edge_index[0].astype(jnp.int32)
    col = edge_index[1].astype(jnp.int32)
    pad = E_PAD - N_EDGES
    row = jnp.pad(row, (0, pad))
    col = jnp.pad(col, (0, pad))
    ew = jnp.pad(edge_weight.astype(jnp.float32), (0, pad))
    epk = jnp.stack([
        row.reshape(-1, CHUNK),
        col.reshape(-1, CHUNK),
        lax.bitcast_convert_type(ew, jnp.int32).reshape(-1, CHUNK),
    ], axis=1)

    h = _matmul(feat, W.T)
    partial = _edge_scatter(h, epk)
    return _combine(partial, prelu_w)


# speedup vs baseline: 1.2268x; 1.0032x over previous
import dataclasses

import jax
import jax.numpy as jnp
from jax import lax
from jax.experimental import pallas as pl
from jax.experimental.pallas import tpu as pltpu
from jax.experimental.pallas import tpu_sc as plsc

N_NODES = 10000
FEAT = 128
N_EDGES = 320000

NC = 2
NS = 16
LANES = 16

CHUNK = 64
K_CHUNKS = 160
NBUF = 5
LAG_G = 2
LAG_I = 3
E_PAD = CHUNK * K_CHUNKS * NC * NS
ACC_N = 10112
ROWS_PER_SUBCORE = ACC_N // NS


def _matmul_body(f_ref, wt_ref, o_ref):
    o_ref[...] = jnp.dot(f_ref[...], wt_ref[...],
                         preferred_element_type=jnp.float32)


def _matmul(feat, Wt):
    blk = 1000
    return pl.pallas_call(
        _matmul_body,
        grid=(N_NODES // blk,),
        in_specs=[
            pl.BlockSpec((blk, FEAT), lambda i: (i, 0)),
            pl.BlockSpec((FEAT, FEAT), lambda i: (0, 0)),
        ],
        out_specs=pl.BlockSpec((blk, FEAT), lambda i: (i, 0)),
        out_shape=jax.ShapeDtypeStruct((N_NODES, FEAT), jnp.float32),
    )(feat, Wt)


def _edge_body(h_hbm, epk_hbm, out_hbm,
               ib0, ib1, ib2, ib3, ib4,
               mb0, mb1, mb2, mb3, mb4,
               sc0, sc1, sc2, sc3, sc4,
               acc,
               si0, si1, si2, si3, si4,
               sg0, sg1, sg2, sg3, sg4,
               ss0, ss1, ss2, ss3, ss4):
    core = lax.axis_index("c")
    sid = lax.axis_index("s")
    wid = core * NS + sid
    cbase = wid * K_CHUNKS

    ib = (ib0, ib1, ib2, ib3, ib4)
    mb = (mb0, mb1, mb2, mb3, mb4)
    scol = (sc0, sc1, sc2, sc3, sc4)
    si = (si0, si1, si2, si3, si4)
    sg = (sg0, sg1, sg2, sg3, sg4)
    ss = (ss0, ss1, ss2, ss3, ss4)

    def fetch_idx(g, m):
        pltpu.async_copy(epk_hbm.at[cbase + g], ib[m], si[m])

    def wait_idx(g, m):
        pltpu.make_async_copy(epk_hbm.at[cbase + g], ib[m], si[m]).wait()

    def issue_gather(m):
        pltpu.async_copy(h_hbm.at[ib[m].at[0]], mb[m], sg[m])

    def wait_msg_bytes(m, sem):
        pltpu.make_async_copy(h_hbm.at[pl.ds(0, CHUNK)], mb[m], sem[m]).wait()

    @pl.loop(0, CHUNK)
    def _(i):
        @pl.loop(0, FEAT, step=LANES)
        def _(j):
            mb0[i, pl.ds(j, LANES)] = jnp.zeros((LANES,), jnp.float32)

    rbase = sid * ROWS_PER_SUBCORE

    @pl.loop(0, 576, step=CHUNK)
    def _(r):
        pltpu.sync_copy(mb0, acc.at[pl.ds(rbase + r, CHUNK)])

    pltpu.sync_copy(mb0.at[pl.ds(0, 56)], acc.at[pl.ds(rbase + 576, 56)])

    for k in range(LAG_I):
        fetch_idx(k, k)
    for k in range(LAG_G):
        wait_idx(k, k)
        issue_gather(k)

    plsc.subcore_barrier()

    def step(g, m):
        m2 = (m + LAG_I) % NBUF

        @pl.when(g + LAG_I < K_CHUNKS)
        def _():
            @pl.when(g + LAG_I >= NBUF)
            def _():
                wait_msg_bytes(m2, ss)
            fetch_idx(g + LAG_I, m2)

        m3 = (m + LAG_G) % NBUF

        @pl.when(g + LAG_G < K_CHUNKS)
        def _():
            wait_idx(g + LAG_G, m3)
            issue_gather(m3)

        wait_msg_bytes(m, sg)

        for k in range(CHUNK // LANES):
            sl = pl.ds(k * LANES, LANES)
            scol[m][sl] = ib[m][1, sl]

        @pl.loop(0, CHUNK, step=LANES)
        def _(e0):
            w16 = plsc.bitcast(ib[m][2, pl.ds(e0, LANES)], jnp.float32)
            for l in range(LANES):
                wvec = jnp.full((LANES,), w16[l], jnp.float32)
                for j in range(FEAT // LANES):
                    sl = pl.ds(j * LANES, LANES)
                    mb[m][e0 + l, sl] = mb[m][e0 + l, sl] * wvec

        pltpu.async_copy(mb[m], acc.at[scol[m]], ss[m], add=True)

    @pl.loop(0, K_CHUNKS, step=NBUF)
    def _(g):
        for k in range(NBUF):
            step(g + k, k)

    for m in range(NBUF):
        wait_msg_bytes(m, ss)
    plsc.subcore_barrier()

    @pl.loop(0, 576, step=CHUNK)
    def _(r):
        pltpu.sync_copy(acc.at[pl.ds(rbase + r, CHUNK)],
                        out_hbm.at[core, pl.ds(rbase + r, CHUNK)])

    pltpu.sync_copy(acc.at[pl.ds(rbase + 576, 56)],
                    out_hbm.at[core, pl.ds(rbase + 576, 56)])


def _edge_scatter(h, epk):
    mesh = plsc.VectorSubcoreMesh(core_axis_name="c", subcore_axis_name="s")
    cp = pltpu.CompilerParams()
    if "needs_layout_passes" in pltpu.CompilerParams.__dataclass_fields__:
        cp = dataclasses.replace(cp, needs_layout_passes=False)
    scratch = (
        [pltpu.VMEM((3, CHUNK), jnp.int32) for _ in range(NBUF)]
        + [pltpu.VMEM((CHUNK, FEAT), jnp.float32) for _ in range(NBUF)]
        + [pltpu.VMEM((CHUNK,), jnp.int32) for _ in range(NBUF)]
        + [pltpu.VMEM_SHARED((ACC_N, FEAT), jnp.float32)]
        + [pltpu.SemaphoreType.DMA] * (3 * NBUF)
    )
    kern = pl.kernel(
        _edge_body,
        compiler_params=cp,
        out_type=jax.ShapeDtypeStruct((NC, ACC_N, FEAT), jnp.float32),
        mesh=mesh,
        scratch_types=scratch,
    )
    return kern(h, epk)


def _combine_body(p_ref, a_ref, o_ref):
    s = p_ref[0] + p_ref[1]
    o_ref[...] = jnp.where(s >= 0, s, a_ref[0] * s)


def _combine(partial, prelu_w):
    blk = 1000
    return pl.pallas_call(
        _combine_body,
        grid=(N_NODES // blk,),
        in_specs=[
            pl.BlockSpec((NC, blk, FEAT), lambda i: (0, i, 0)),
            pl.BlockSpec(memory_space=pltpu.SMEM),
        ],
        out_specs=pl.BlockSpec((blk, FEAT), lambda i: (i, 0)),
        out_shape=jax.ShapeDtypeStruct((N_NODES, FEAT), jnp.float32),
    )(partial, prelu_w.reshape(1))


def kernel(feat, edge_index, edge_weight, W, prelu_w):
    row = edge_index[0].astype(jnp.int32)
    col = edge_index[1].astype(jnp.int32)
    pad = E_PAD - N_EDGES
    row = jnp.pad(row, (0, pad))
    col = jnp.pad(col, (0, pad))
    ew = jnp.pad(edge_weight.astype(jnp.float32), (0, pad))
    epk = jnp.stack([
        row.reshape(-1, CHUNK),
        col.reshape(-1, CHUNK),
        lax.bitcast_convert_type(ew, jnp.int32).reshape(-1, CHUNK),
    ], axis=1)

    h = _matmul(feat, W.T)
    partial = _edge_scatter(h, epk)
    return _combine(partial, prelu_w)
